# baseline (speedup 1.0000x reference)
import functools

import jax
import jax.numpy as jnp
from jax import lax
from jax.experimental import pallas as pl
from jax.experimental.pallas import tpu as pltpu
from jax.experimental.pallas import tpu_sc as plsc

_NUM_OBJ = 11
_B = 1024
_NV = 8648
_NGRP = _NV // 16
_TAIL_OFF = _NV - 16


def _prep_params_body(th_ref, go_ref, tr_ref, sc_ref, out_ref):
    th = th_ref[0:1, :]
    c = jnp.cos(th)
    sn = jnp.sin(th)
    gx = go_ref[0:1, :]
    gy = go_ref[1:2, :]
    gz = go_ref[2:3, :]
    n2 = gx * gx + gy * gy + gz * gz
    n = jnp.sqrt(n2)
    half = 0.5 * n
    small = n < 1e-6
    safe = jnp.where(small, jnp.ones_like(n), n)
    soa = jnp.where(small, 0.5 - n2 / 48.0, jnp.sin(half) / safe)
    w = jnp.cos(half)
    x = gx * soa
    y = gy * soa
    z = gz * soa
    g00 = 1.0 - 2.0 * (y * y + z * z)
    g01 = 2.0 * (x * y - w * z)
    g02 = 2.0 * (x * z + w * y)
    g10 = 2.0 * (x * y + w * z)
    g11 = 1.0 - 2.0 * (x * x + z * z)
    g12 = 2.0 * (y * z - w * x)
    g20 = 2.0 * (x * z - w * y)
    g21 = 2.0 * (y * z + w * x)
    g22 = 1.0 - 2.0 * (x * x + y * y)
    out_ref[...] = jnp.concatenate(
        [g00, g01, g02, g10, g11, g12, g20, g21, g22,
         tr_ref[0:1, :], tr_ref[1:2, :], tr_ref[2:3, :],
         sc_ref[0:1, :] - 1.0, c, sn, -sn],
        axis=0,
    )


def _prep_params(angles, global_orient, transl, obj_scale):
    th = angles.reshape(1, _B)
    go = global_orient.T.reshape(3, _B)
    tr = transl.T.reshape(3, _B)
    sc = obj_scale.reshape(1, _B)
    return pl.pallas_call(
        _prep_params_body,
        out_shape=jax.ShapeDtypeStruct((16, _B), jnp.float32),
    )(th, go, tr, sc)


def _sc_transform(tab, gflag, params, obj_idx):
    info = plsc.get_sparse_core_info()
    nc, ns = info.num_cores, info.num_subcores
    nw = nc * ns
    rows = _B // nw

    mesh = plsc.VectorSubcoreMesh(core_axis_name="c", subcore_axis_name="s")

    @functools.partial(
        pl.kernel,
        mesh=mesh,
        out_type=jax.ShapeDtypeStruct((_B, _NV * 3), jnp.float32),
        scratch_types=[
            pltpu.VMEM((rows, 16), jnp.float32),
            pltpu.VMEM((rows, 16), jnp.int32),
            pltpu.VMEM((4, _NV), jnp.float32),
            pltpu.VMEM((_NV,), jnp.float32),
            pltpu.VMEM((_NV * 3,), jnp.float32),
        ],
    )
    def k(tab_hbm, g_hbm, params_hbm, obj_hbm, out_hbm,
          params_v, obj_v, tab_v, g_v, stage_v):
        wid = lax.axis_index("s") * nc + lax.axis_index("c")
        b0 = wid * rows
        pltpu.sync_copy(params_hbm.at[pl.ds(b0, rows), :], params_v)
        pltpu.sync_copy(obj_hbm.at[pl.ds(b0, rows), :], obj_v)
        pltpu.sync_copy(g_hbm, g_v)

        def row_body(i, carry):
            obj = obj_v[i, :][0]
            pltpu.sync_copy(tab_hbm.at[obj], tab_v)
            pv = params_v[i, :]
            g00 = pv[0]
            g01 = pv[1]
            g02 = pv[2]
            g10 = pv[3]
            g11 = pv[4]
            g12 = pv[5]
            g20 = pv[6]
            g21 = pv[7]
            g22 = pv[8]
            tx = pv[9]
            ty = pv[10]
            tz = pv[11]
            sm1 = pv[12]
            cth = pv[13]
            snt = pv[14]
            nsnt = pv[15]

            def grp(off):
                px = tab_v[0, pl.ds(off, 16)]
                py = tab_v[1, pl.ds(off, 16)]
                pz = tab_v[2, pl.ds(off, 16)]
                m = tab_v[3, pl.ds(off, 16)]
                gv = g_v[pl.ds(off, 16)]
                sef = gv * sm1 + 1.0
                ux = px * sef
                uy = py * sef
                uz = pz * sef
                ax = ux * cth + uy * snt
                ay = ux * nsnt + uy * cth
                wx = ux + m * (ax - ux)
                wy = uy + m * (ay - uy)
                ox = g00 * wx + g01 * wy + g02 * uz + tx * gv
                oy = g10 * wx + g11 * wy + g12 * uz + ty * gv
                oz = g20 * wx + g21 * wy + g22 * uz + tz * gv
                base = off * 3
                lane = lax.broadcasted_iota(jnp.int32, (16,), 0)
                for w_ in range(3):
                    pos = lane + (16 * w_)
                    j = lax.shift_right_logical(pos * 21846, 16)
                    cc = pos - j * 3
                    ga = ox.at[j].get(mode="promise_in_bounds")
                    gb = oy.at[j].get(mode="promise_in_bounds")
                    gc = oz.at[j].get(mode="promise_in_bounds")
                    word = jnp.where(cc == 0, ga, jnp.where(cc == 1, gb, gc))
                    stage_v[pl.ds(base + 16 * w_, 16)] = word

            lax.fori_loop(0, _NGRP, lambda gi, c: (grp(gi * 16), c)[1], 0)
            grp(_TAIL_OFF)
            pltpu.sync_copy(stage_v, out_hbm.at[b0 + i])
            return carry

        lax.fori_loop(0, rows, row_body, 0)

    return k(tab, gflag, params, obj_idx.astype(jnp.int32))


def kernel(angles, global_orient, transl, obj_scale, obj_idx, table_v,
           table_v_normal, table_v_sub, table_bbox_top, table_bbox_bottom,
           table_kp_top, table_kp_bottom, table_parts_ids,
           table_parts_sub_ids):
    f32 = jnp.float32
    xyz = jnp.concatenate(
        [table_v, table_v_normal, table_v_sub, table_bbox_top,
         table_bbox_bottom, table_kp_top, table_kp_bottom], axis=1)
    xyz = jnp.transpose(xyz, (0, 2, 1))
    mask = jnp.concatenate(
        [(table_parts_ids == 1).astype(f32),
         (table_parts_ids == 1).astype(f32),
         (table_parts_sub_ids == 1).astype(f32),
         jnp.ones((_NUM_OBJ, 8), f32),
         jnp.zeros((_NUM_OBJ, 8), f32),
         jnp.ones((_NUM_OBJ, 16), f32),
         jnp.zeros((_NUM_OBJ, 16), f32)], axis=1)[:, None, :]
    tab = jnp.concatenate([xyz, mask], axis=1)
    gflag = jnp.concatenate(
        [jnp.ones((4000,), f32), jnp.zeros((4000,), f32),
         jnp.ones((_NV - 8000,), f32)])
    params = _prep_params(angles, global_orient, transl, obj_scale).T
    objb = jnp.broadcast_to(obj_idx.astype(jnp.int32)[:, None], (_B, 16))
    out = _sc_transform(tab, gflag, params, objb)
    return out.reshape(_B, _NV, 3)


# speedup vs baseline: 1.1532x; 1.1532x over previous
import functools

import jax
import jax.numpy as jnp
from jax import lax
from jax.experimental import pallas as pl
from jax.experimental.pallas import tpu as pltpu
from jax.experimental.pallas import tpu_sc as plsc

_NUM_OBJ = 11
_B = 1024
_NV = 8648
_TAIL_OFF = _NV - 16


def _prep_params_body(th_ref, go_ref, tr_ref, sc_ref, out_ref):
    th = th_ref[0:1, :]
    c = jnp.cos(th)
    sn = jnp.sin(th)
    gx = go_ref[0:1, :]
    gy = go_ref[1:2, :]
    gz = go_ref[2:3, :]
    n2 = gx * gx + gy * gy + gz * gz
    n = jnp.sqrt(n2)
    half = 0.5 * n
    small = n < 1e-6
    safe = jnp.where(small, jnp.ones_like(n), n)
    soa = jnp.where(small, 0.5 - n2 / 48.0, jnp.sin(half) / safe)
    w = jnp.cos(half)
    x = gx * soa
    y = gy * soa
    z = gz * soa
    g00 = 1.0 - 2.0 * (y * y + z * z)
    g01 = 2.0 * (x * y - w * z)
    g02 = 2.0 * (x * z + w * y)
    g10 = 2.0 * (x * y + w * z)
    g11 = 1.0 - 2.0 * (x * x + z * z)
    g12 = 2.0 * (y * z - w * x)
    g20 = 2.0 * (x * z - w * y)
    g21 = 2.0 * (y * z + w * x)
    g22 = 1.0 - 2.0 * (x * x + y * y)
    s = sc_ref[0:1, :]
    zero = jnp.zeros_like(s)
    out_ref[...] = jnp.concatenate(
        [s * g00, s * g01, s * g02, s * g10, s * g11, s * g12,
         s * g20, s * g21, s * g22,
         tr_ref[0:1, :], tr_ref[1:2, :], tr_ref[2:3, :],
         c, sn, -sn, zero,
         g00, g01, g02, g10, g11, g12, g20, g21, g22,
         zero, zero, zero, zero, zero, zero, zero],
        axis=0,
    )


def _prep_params(angles, global_orient, transl, obj_scale):
    th = angles.reshape(1, _B)
    go = global_orient.T.reshape(3, _B)
    tr = transl.T.reshape(3, _B)
    sc = obj_scale.reshape(1, _B)
    return pl.pallas_call(
        _prep_params_body,
        out_shape=jax.ShapeDtypeStruct((32, _B), jnp.float32),
    )(th, go, tr, sc)


def _sc_transform(tab, params, obj_idx):
    info = plsc.get_sparse_core_info()
    nc, ns = info.num_cores, info.num_subcores
    nw = nc * ns
    rows = _B // nw

    mesh = plsc.VectorSubcoreMesh(core_axis_name="c", subcore_axis_name="s")

    @functools.partial(
        pl.kernel,
        mesh=mesh,
        out_type=jax.ShapeDtypeStruct((_B, _NV * 3), jnp.float32),
        scratch_types=[
            pltpu.VMEM((rows, 32), jnp.float32),
            pltpu.VMEM((rows, 16), jnp.int32),
            pltpu.VMEM((4, _NV), jnp.float32),
            pltpu.VMEM((4, _NV), jnp.float32),
            pltpu.VMEM((_NV * 3,), jnp.float32),
            pltpu.VMEM((_NV * 3,), jnp.float32),
            pltpu.SemaphoreType.DMA,
            pltpu.SemaphoreType.DMA,
            pltpu.SemaphoreType.DMA,
            pltpu.SemaphoreType.DMA,
        ],
    )
    def k(tab_hbm, params_hbm, obj_hbm, out_hbm,
          params_v, obj_v, taba_v, tabb_v, stg0_v, stg1_v,
          semta, semtb, sems0, sems1):
        wid = lax.axis_index("s") * nc + lax.axis_index("c")
        b0 = wid * rows
        pltpu.sync_copy(params_hbm.at[pl.ds(b0, rows), :], params_v)
        pltpu.sync_copy(obj_hbm.at[pl.ds(b0, rows), :], obj_v)

        lane = lax.broadcasted_iota(jnp.int32, (16,), 0)
        idxj = []
        ccs = []
        for w_ in range(3):
            pos = lane + (16 * w_)
            j = lax.shift_right_logical(pos * 21846, 16)
            idxj.append(j)
            ccs.append(pos - j * 3)

        def bcast(x):
            return jnp.full((16,), x, jnp.float32)

        def compute_row(i, tab_v, stage_v):
            pv1 = params_v[i, pl.ds(0, 16)]
            pv2 = params_v[i, pl.ds(16, 16)]
            sa = [bcast(pv1[q]) for q in range(9)]
            tv = [bcast(pv1[q]) for q in range(9, 12)]
            cv = bcast(pv1[12])
            snv = bcast(pv1[13])
            nsnv = bcast(pv1[14])
            na = [bcast(pv2[q]) for q in range(9)]

            def grp(off, aa, with_t):
                px = tab_v[0, pl.ds(off, 16)]
                py = tab_v[1, pl.ds(off, 16)]
                pz = tab_v[2, pl.ds(off, 16)]
                m = tab_v[3, pl.ds(off, 16)]
                ax = px * cv + py * snv
                ay = px * nsnv + py * cv
                wx = px + m * (ax - px)
                wy = py + m * (ay - py)
                if with_t:
                    ox = aa[0] * wx + aa[1] * wy + (aa[2] * pz + tv[0])
                    oy = aa[3] * wx + aa[4] * wy + (aa[5] * pz + tv[1])
                    oz = aa[6] * wx + aa[7] * wy + (aa[8] * pz + tv[2])
                else:
                    ox = aa[0] * wx + aa[1] * wy + aa[2] * pz
                    oy = aa[3] * wx + aa[4] * wy + aa[5] * pz
                    oz = aa[6] * wx + aa[7] * wy + aa[8] * pz
                base = off * 3
                for w_ in range(3):
                    ga = ox.at[idxj[w_]].get(mode="promise_in_bounds")
                    gb = oy.at[idxj[w_]].get(mode="promise_in_bounds")
                    gc = oz.at[idxj[w_]].get(mode="promise_in_bounds")
                    word = jnp.where(ccs[w_] == 0, ga,
                                     jnp.where(ccs[w_] == 1, gb, gc))
                    stage_v[pl.ds(base + 16 * w_, 16)] = word

            def segment(v0, ngrp, nun, aa, with_t):
                def body(q, c):
                    for u in range(nun):
                        grp(v0 + (q * nun + u) * 16, aa, with_t)
                    return c
                lax.fori_loop(0, ngrp // nun, body, 0)

            segment(0, 250, 5, sa, True)
            segment(4000, 250, 5, na, False)
            segment(8000, 40, 5, sa, True)
            grp(_TAIL_OFF, sa, True)

        def tab_copy(i, buf, sem):
            ov = obj_v[i, :]
            return pltpu.make_async_copy(tab_hbm.at[ov[0]], buf, sem)

        tab_copy(0, taba_v, semta).start()

        nk = rows // 2

        def body(kk, carry):
            i0 = 2 * kk
            i1 = i0 + 1
            tab_copy(i0, taba_v, semta).wait()
            tab_copy(i1, tabb_v, semtb).start()

            @pl.when(kk >= 1)
            def _():
                pltpu.make_async_copy(
                    stg0_v, out_hbm.at[b0 + i0 - 2], sems0).wait()

            compute_row(i0, taba_v, stg0_v)
            pltpu.make_async_copy(stg0_v, out_hbm.at[b0 + i0], sems0).start()

            @pl.when(kk < nk - 1)
            def _():
                tab_copy(i0 + 2, taba_v, semta).start()

            tab_copy(i1, tabb_v, semtb).wait()

            @pl.when(kk >= 1)
            def _():
                pltpu.make_async_copy(
                    stg1_v, out_hbm.at[b0 + i1 - 2], sems1).wait()

            compute_row(i1, tabb_v, stg1_v)
            pltpu.make_async_copy(stg1_v, out_hbm.at[b0 + i1], sems1).start()
            return carry

        lax.fori_loop(0, nk, body, 0)
        pltpu.make_async_copy(stg0_v, out_hbm.at[b0 + rows - 2], sems0).wait()
        pltpu.make_async_copy(stg1_v, out_hbm.at[b0 + rows - 1], sems1).wait()

    return k(tab, params, obj_idx)


def kernel(angles, global_orient, transl, obj_scale, obj_idx, table_v,
           table_v_normal, table_v_sub, table_bbox_top, table_bbox_bottom,
           table_kp_top, table_kp_bottom, table_parts_ids,
           table_parts_sub_ids):
    f32 = jnp.float32
    xyz = jnp.concatenate(
        [table_v, table_v_normal, table_v_sub, table_bbox_top,
         table_bbox_bottom, table_kp_top, table_kp_bottom], axis=1)
    xyz = jnp.transpose(xyz, (0, 2, 1))
    mask = jnp.concatenate(
        [(table_parts_ids == 1).astype(f32),
         (table_parts_ids == 1).astype(f32),
         (table_parts_sub_ids == 1).astype(f32),
         jnp.ones((_NUM_OBJ, 8), f32),
         jnp.zeros((_NUM_OBJ, 8), f32),
         jnp.ones((_NUM_OBJ, 16), f32),
         jnp.zeros((_NUM_OBJ, 16), f32)], axis=1)[:, None, :]
    tab = jnp.concatenate([xyz, mask], axis=1)
    params = _prep_params(angles, global_orient, transl, obj_scale).T
    objb = jnp.broadcast_to(obj_idx.astype(jnp.int32)[:, None], (_B, 16))
    out = _sc_transform(tab, params, objb)
    return out.reshape(_B, _NV, 3)


# speedup vs baseline: 6.2202x; 5.3938x over previous
import functools

import jax
import jax.numpy as jnp
from jax import lax
from jax.experimental import pallas as pl
from jax.experimental.pallas import tpu as pltpu
from jax.experimental.pallas import tpu_sc as plsc

_NUM_OBJ = 11
_B = 1024
_NV = 8648
_WLEN = 336
_NG = _B // 16


def _prep_params_body(th_ref, go_ref, tr_ref, sc_ref, out_ref):
    th = th_ref[0:1, :]
    c = jnp.cos(th)
    sn = jnp.sin(th)
    gx = go_ref[0:1, :]
    gy = go_ref[1:2, :]
    gz = go_ref[2:3, :]
    n2 = gx * gx + gy * gy + gz * gz
    n = jnp.sqrt(n2)
    half = 0.5 * n
    small = n < 1e-6
    safe = jnp.where(small, jnp.ones_like(n), n)
    soa = jnp.where(small, 0.5 - n2 / 48.0, jnp.sin(half) / safe)
    w = jnp.cos(half)
    x = gx * soa
    y = gy * soa
    z = gz * soa
    g00 = 1.0 - 2.0 * (y * y + z * z)
    g01 = 2.0 * (x * y - w * z)
    g02 = 2.0 * (x * z + w * y)
    g10 = 2.0 * (x * y + w * z)
    g11 = 1.0 - 2.0 * (x * x + z * z)
    g12 = 2.0 * (y * z - w * x)
    g20 = 2.0 * (x * z - w * y)
    g21 = 2.0 * (y * z + w * x)
    g22 = 1.0 - 2.0 * (x * x + y * y)
    s = sc_ref[0:1, :]
    zero = jnp.zeros_like(s)
    out_ref[...] = jnp.concatenate(
        [s * g00, s * g01, s * g02, s * g10, s * g11, s * g12,
         s * g20, s * g21, s * g22,
         tr_ref[0:1, :], tr_ref[1:2, :], tr_ref[2:3, :],
         c, sn, -sn, zero,
         g00, g01, g02, g10, g11, g12, g20, g21, g22,
         zero, zero, zero, zero, zero, zero, zero],
        axis=0,
    )


def _prep_params(angles, global_orient, transl, obj_scale):
    th = angles.reshape(1, _B)
    go = global_orient.T.reshape(3, _B)
    tr = transl.T.reshape(3, _B)
    sc = obj_scale.reshape(1, _B)
    return pl.pallas_call(
        _prep_params_body,
        out_shape=jax.ShapeDtypeStruct((32, _B), jnp.float32),
    )(th, go, tr, sc)


def _worker_starts():
    starts = []
    for w in range(15):
        starts.append(min(272 * w, 3728))
    for w in range(15):
        starts.append(4000 + min(272 * w, 3728))
    starts.append(8000)
    starts.append(8312)
    return starts


def _sc_transform(tabw, params, objs):
    info = plsc.get_sparse_core_info()
    nc, ns = info.num_cores, info.num_subcores

    mesh = plsc.VectorSubcoreMesh(core_axis_name="c", subcore_axis_name="s")

    @functools.partial(
        pl.kernel,
        mesh=mesh,
        out_type=jax.ShapeDtypeStruct((3, _NV, _B), jnp.float32),
        scratch_types=[
            pltpu.VMEM((32, _B), jnp.float32),
            pltpu.VMEM((_B,), jnp.int32),
            pltpu.VMEM((3, _WLEN * 16), jnp.int32),
            pltpu.VMEM((3, 8, _B), jnp.float32),
            pltpu.VMEM((3, 8, _B), jnp.float32),
            pltpu.SemaphoreType.DMA,
            pltpu.SemaphoreType.DMA,
        ],
    )
    def k(tabw_hbm, params_hbm, objs_hbm, out_hbm,
          params_v, objs_v, tabw_v, stg0_v, stg1_v, sems0, sems1):
        wid = lax.axis_index("s") * nc + lax.axis_index("c")
        pltpu.sync_copy(params_hbm, params_v)
        pltpu.sync_copy(objs_hbm, objs_v)
        pltpu.sync_copy(tabw_hbm.at[wid], tabw_v)

        def do_chunk(v0, ch, stage_v, scaled):
            @plsc.parallel_loop(0, _NG, 1, unroll=2)
            def gbody(g):
                b0 = g * 16
                obj16 = objs_v[pl.ds(b0, 16)]
                if scaled:
                    a = [params_v[r, pl.ds(b0, 16)] for r in range(9)]
                    tvec = [params_v[r, pl.ds(b0, 16)] for r in (9, 10, 11)]
                else:
                    a = [params_v[16 + r, pl.ds(b0, 16)] for r in range(9)]
                    tvec = None
                cthv = params_v[12, pl.ds(b0, 16)]
                sntv = params_v[13, pl.ds(b0, 16)]
                nsntv = params_v[14, pl.ds(b0, 16)]
                for vl in range(8):
                    v_ = ch * 8 + vl
                    gi = lambda c_: (
                        tabw_v[c_, pl.ds(v_ * 16, 16)]
                        .at[obj16].get(mode="promise_in_bounds"))
                    pxi = gi(0)
                    bc = lambda x: lax.bitcast_convert_type(x, jnp.float32)
                    px = bc(pxi)
                    py = bc(gi(1))
                    pz = bc(gi(2))
                    mm = (pxi & 1) == 1
                    ax = px * cthv + py * sntv
                    ay = px * nsntv + py * cthv
                    wx = jnp.where(mm, ax, px)
                    wy = jnp.where(mm, ay, py)
                    if scaled:
                        ox = a[0] * wx + a[1] * wy + (a[2] * pz + tvec[0])
                        oy = a[3] * wx + a[4] * wy + (a[5] * pz + tvec[1])
                        oz = a[6] * wx + a[7] * wy + (a[8] * pz + tvec[2])
                    else:
                        ox = a[0] * wx + a[1] * wy + a[2] * pz
                        oy = a[3] * wx + a[4] * wy + a[5] * pz
                        oz = a[6] * wx + a[7] * wy + a[8] * pz
                    stage_v[0, vl, pl.ds(b0, 16)] = ox
                    stage_v[1, vl, pl.ds(b0, 16)] = oy
                    stage_v[2, vl, pl.ds(b0, 16)] = oz

        def variant(v0, nch, scaled):
            nk = nch // 2

            def body(kk, carry):
                ch0 = 2 * kk
                ch1 = ch0 + 1

                @pl.when(kk >= 1)
                def _():
                    pltpu.make_async_copy(
                        stg0_v, out_hbm.at[:, pl.ds(v0, 8), :], sems0).wait()

                do_chunk(v0, ch0, stg0_v, scaled)
                pltpu.make_async_copy(
                    stg0_v, out_hbm.at[:, pl.ds(v0 + ch0 * 8, 8), :],
                    sems0).start()

                @pl.when(kk >= 1)
                def _():
                    pltpu.make_async_copy(
                        stg1_v, out_hbm.at[:, pl.ds(v0, 8), :], sems1).wait()

                do_chunk(v0, ch1, stg1_v, scaled)
                pltpu.make_async_copy(
                    stg1_v, out_hbm.at[:, pl.ds(v0 + ch1 * 8, 8), :],
                    sems1).start()
                return carry

            lax.fori_loop(0, nk, body, 0)
            pltpu.make_async_copy(
                stg0_v, out_hbm.at[:, pl.ds(v0, 8), :], sems0).wait()
            pltpu.make_async_copy(
                stg1_v, out_hbm.at[:, pl.ds(v0, 8), :], sems1).wait()

        wv = jnp.minimum(272 * wid, 3728)

        @pl.when(wid < 15)
        def _():
            variant(wv, 34, True)

        wb = 4000 + jnp.minimum(272 * (wid - 15), 3728)

        @pl.when((wid >= 15) & (wid < 30))
        def _():
            variant(wb, 34, False)

        wc = 8000 + (wid - 30) * 312

        @pl.when(wid >= 30)
        def _():
            variant(wc, 42, True)

    return k(tabw, params, objs)


def kernel(angles, global_orient, transl, obj_scale, obj_idx, table_v,
           table_v_normal, table_v_sub, table_bbox_top, table_bbox_bottom,
           table_kp_top, table_kp_bottom, table_parts_ids,
           table_parts_sub_ids):
    i32 = jnp.int32
    xyz = jnp.concatenate(
        [table_v, table_v_normal, table_v_sub, table_bbox_top,
         table_bbox_bottom, table_kp_top, table_kp_bottom], axis=1)
    xyz = jnp.transpose(xyz, (0, 2, 1))
    mask = jnp.concatenate(
        [(table_parts_ids == 1).astype(i32),
         (table_parts_ids == 1).astype(i32),
         (table_parts_sub_ids == 1).astype(i32),
         jnp.ones((_NUM_OBJ, 8), i32),
         jnp.zeros((_NUM_OBJ, 8), i32),
         jnp.ones((_NUM_OBJ, 16), i32),
         jnp.zeros((_NUM_OBJ, 16), i32)], axis=1)
    tabi = jax.lax.bitcast_convert_type(xyz, i32)
    x_lsb = (tabi[:, 0, :] & ~1) | mask
    tab = tabi.at[:, 0, :].set(x_lsb)
    tabp = jnp.pad(tab, ((0, 5), (0, 0), (0, _WLEN)))
    wins = [jnp.transpose(tabp[:, :, v0:v0 + _WLEN], (1, 2, 0))
            for v0 in _worker_starts()]
    tabw = jnp.stack(wins, axis=0).reshape(32, 3, _WLEN * 16)
    params = _prep_params(angles, global_orient, transl, obj_scale)
    out = _sc_transform(tabw, params, obj_idx.astype(i32))
    return jnp.transpose(out, (2, 1, 0))


# speedup vs baseline: 6.7933x; 1.0921x over previous
import functools

import jax
import jax.numpy as jnp
from jax import lax
from jax.experimental import pallas as pl
from jax.experimental.pallas import tpu as pltpu
from jax.experimental.pallas import tpu_sc as plsc

_NUM_OBJ = 11
_B = 1024
_NV = 8648
_WLEN = 336
_NG = _B // 16


def _prep_params_body(th_ref, go_ref, tr_ref, sc_ref, out_ref):
    th = th_ref[0:1, :]
    c = jnp.cos(th)
    sn = jnp.sin(th)
    gx = go_ref[0:1, :]
    gy = go_ref[1:2, :]
    gz = go_ref[2:3, :]
    n2 = gx * gx + gy * gy + gz * gz
    n = jnp.sqrt(n2)
    half = 0.5 * n
    small = n < 1e-6
    safe = jnp.where(small, jnp.ones_like(n), n)
    soa = jnp.where(small, 0.5 - n2 / 48.0, jnp.sin(half) / safe)
    w = jnp.cos(half)
    x = gx * soa
    y = gy * soa
    z = gz * soa
    g00 = 1.0 - 2.0 * (y * y + z * z)
    g01 = 2.0 * (x * y - w * z)
    g02 = 2.0 * (x * z + w * y)
    g10 = 2.0 * (x * y + w * z)
    g11 = 1.0 - 2.0 * (x * x + z * z)
    g12 = 2.0 * (y * z - w * x)
    g20 = 2.0 * (x * z - w * y)
    g21 = 2.0 * (y * z + w * x)
    g22 = 1.0 - 2.0 * (x * x + y * y)
    s = sc_ref[0:1, :]
    zero = jnp.zeros_like(s)
    out_ref[...] = jnp.concatenate(
        [s * g00, s * g01, s * g02, s * g10, s * g11, s * g12,
         s * g20, s * g21, s * g22,
         tr_ref[0:1, :], tr_ref[1:2, :], tr_ref[2:3, :],
         c, sn, -sn, zero,
         g00, g01, g02, g10, g11, g12, g20, g21, g22,
         zero, zero, zero, zero, zero, zero, zero],
        axis=0,
    )


def _prep_params(angles, global_orient, transl, obj_scale):
    th = angles.reshape(1, _B)
    go = global_orient.T.reshape(3, _B)
    tr = transl.T.reshape(3, _B)
    sc = obj_scale.reshape(1, _B)
    return pl.pallas_call(
        _prep_params_body,
        out_shape=jax.ShapeDtypeStruct((32, _B), jnp.float32),
    )(th, go, tr, sc)


def _worker_starts():
    starts = []
    for w in range(15):
        starts.append(min(272 * w, 3728))
    for w in range(14):
        starts.append(4000 + min(288 * w, 3712))
    starts.extend([8000, 8208, 8424])
    return starts


def _sc_transform(tabw, params, objs):
    info = plsc.get_sparse_core_info()
    nc, ns = info.num_cores, info.num_subcores

    mesh = plsc.VectorSubcoreMesh(core_axis_name="c", subcore_axis_name="s")

    @functools.partial(
        pl.kernel,
        mesh=mesh,
        out_type=jax.ShapeDtypeStruct((3, _NV, _B), jnp.float32),
        scratch_types=[
            pltpu.VMEM((32, _B), jnp.float32),
            pltpu.VMEM((_B,), jnp.int32),
            pltpu.VMEM((3, _WLEN * 16), jnp.int32),
            pltpu.VMEM((3, 8, _B), jnp.float32),
            pltpu.VMEM((3, 8, _B), jnp.float32),
            pltpu.SemaphoreType.DMA,
            pltpu.SemaphoreType.DMA,
        ],
    )
    def k(tabw_hbm, params_hbm, objs_hbm, out_hbm,
          params_v, objs_v, tabw_v, stg0_v, stg1_v, sems0, sems1):
        wid = lax.axis_index("s") * nc + lax.axis_index("c")
        pltpu.sync_copy(params_hbm, params_v)
        pltpu.sync_copy(objs_hbm, objs_v)
        pltpu.sync_copy(tabw_hbm.at[wid], tabw_v)

        def do_chunk(v0, ch, stage_v, scaled):
            @plsc.parallel_loop(0, _NG, 1, unroll=2)
            def gbody(g):
                b0 = g * 16
                obj16 = objs_v[pl.ds(b0, 16)]
                if scaled:
                    a = [params_v[r, pl.ds(b0, 16)] for r in range(9)]
                    tvec = [params_v[r, pl.ds(b0, 16)] for r in (9, 10, 11)]
                else:
                    a = [params_v[16 + r, pl.ds(b0, 16)] for r in range(9)]
                    tvec = None
                cthv = params_v[12, pl.ds(b0, 16)]
                sntv = params_v[13, pl.ds(b0, 16)]
                nsntv = params_v[14, pl.ds(b0, 16)]
                for vl in range(8):
                    v_ = ch * 8 + vl
                    gi = lambda c_: (
                        tabw_v[c_, pl.ds(v_ * 16, 16)]
                        .at[obj16].get(mode="promise_in_bounds"))
                    pxi = gi(0)
                    bc = lambda x: lax.bitcast_convert_type(x, jnp.float32)
                    px = bc(pxi)
                    py = bc(gi(1))
                    pz = bc(gi(2))
                    mm = (pxi & 1) == 1
                    ax = px * cthv + py * sntv
                    ay = px * nsntv + py * cthv
                    wx = jnp.where(mm, ax, px)
                    wy = jnp.where(mm, ay, py)
                    if scaled:
                        ox = a[0] * wx + a[1] * wy + (a[2] * pz + tvec[0])
                        oy = a[3] * wx + a[4] * wy + (a[5] * pz + tvec[1])
                        oz = a[6] * wx + a[7] * wy + (a[8] * pz + tvec[2])
                    else:
                        ox = a[0] * wx + a[1] * wy + a[2] * pz
                        oy = a[3] * wx + a[4] * wy + a[5] * pz
                        oz = a[6] * wx + a[7] * wy + a[8] * pz
                    stage_v[0, vl, pl.ds(b0, 16)] = ox
                    stage_v[1, vl, pl.ds(b0, 16)] = oy
                    stage_v[2, vl, pl.ds(b0, 16)] = oz

        def variant(v0, nch, scaled):
            nk = nch // 2

            def body(kk, carry):
                ch0 = 2 * kk
                ch1 = ch0 + 1

                @pl.when(kk >= 1)
                def _():
                    pltpu.make_async_copy(
                        stg0_v, out_hbm.at[:, pl.ds(v0, 8), :], sems0).wait()

                do_chunk(v0, ch0, stg0_v, scaled)
                pltpu.make_async_copy(
                    stg0_v, out_hbm.at[:, pl.ds(v0 + ch0 * 8, 8), :],
                    sems0).start()

                @pl.when(kk >= 1)
                def _():
                    pltpu.make_async_copy(
                        stg1_v, out_hbm.at[:, pl.ds(v0, 8), :], sems1).wait()

                do_chunk(v0, ch1, stg1_v, scaled)
                pltpu.make_async_copy(
                    stg1_v, out_hbm.at[:, pl.ds(v0 + ch1 * 8, 8), :],
                    sems1).start()
                return carry

            lax.fori_loop(0, nk, body, 0)
            pltpu.make_async_copy(
                stg0_v, out_hbm.at[:, pl.ds(v0, 8), :], sems0).wait()
            pltpu.make_async_copy(
                stg1_v, out_hbm.at[:, pl.ds(v0, 8), :], sems1).wait()

        wv = jnp.minimum(272 * wid, 3728)

        @pl.when(wid < 15)
        def _():
            variant(wv, 34, True)

        wb = 4000 + jnp.minimum(288 * (wid - 15), 3712)

        @pl.when((wid >= 15) & (wid < 29))
        def _():
            variant(wb, 36, False)

        wc = 8000 + (wid - 29) * 208 + jnp.maximum(0, wid - 30) * 8

        @pl.when(wid >= 29)
        def _():
            variant(wc, 28, True)

    return k(tabw, params, objs)


def kernel(angles, global_orient, transl, obj_scale, obj_idx, table_v,
           table_v_normal, table_v_sub, table_bbox_top, table_bbox_bottom,
           table_kp_top, table_kp_bottom, table_parts_ids,
           table_parts_sub_ids):
    i32 = jnp.int32
    xyz = jnp.concatenate(
        [table_v, table_v_normal, table_v_sub, table_bbox_top,
         table_bbox_bottom, table_kp_top, table_kp_bottom], axis=1)
    xyz = jnp.transpose(xyz, (0, 2, 1))
    mask = jnp.concatenate(
        [(table_parts_ids == 1).astype(i32),
         (table_parts_ids == 1).astype(i32),
         (table_parts_sub_ids == 1).astype(i32),
         jnp.ones((_NUM_OBJ, 8), i32),
         jnp.zeros((_NUM_OBJ, 8), i32),
         jnp.ones((_NUM_OBJ, 16), i32),
         jnp.zeros((_NUM_OBJ, 16), i32)], axis=1)
    tabi = jax.lax.bitcast_convert_type(xyz, i32)
    x_lsb = (tabi[:, 0, :] & ~1) | mask
    tab = tabi.at[:, 0, :].set(x_lsb)
    tabp = jnp.pad(tab, ((0, 5), (0, 0), (0, _WLEN)))
    wins = [jnp.transpose(tabp[:, :, v0:v0 + _WLEN], (1, 2, 0))
            for v0 in _worker_starts()]
    tabw = jnp.stack(wins, axis=0).reshape(32, 3, _WLEN * 16)
    params = _prep_params(angles, global_orient, transl, obj_scale)
    out = _sc_transform(tabw, params, obj_idx.astype(i32))
    return jnp.transpose(out, (2, 1, 0))
